# baseline (speedup 1.0000x reference)
import jax
import jax.numpy as jnp
from jax.experimental import pallas as pl
from jax.experimental.pallas import tpu as pltpu

N = 8192
D_IN = 768
D_OUT = 768
E = 16
R = 8
SCALING = 16.0 / 8.0

BLOCK_N = 1024


def _fused_kernel(x_ref, w_ref, bias_ref, a_ref, b_ref, ids_ref, out_ref,
                  wb_scratch, a_scratch):
    @pl.when(pl.program_id(0) == 0)
    def _build_weights():
        wb_scratch[:D_IN, :] = w_ref[...]
        wb_scratch[D_IN:, :] = b_ref[...] * SCALING
        af = a_ref[...]
        a3 = af.reshape(E, D_IN, R)
        at = jnp.transpose(a3, (1, 0, 2))
        a_scratch[...] = at.reshape(D_IN, E * R)

    x = x_ref[...]
    lr = jax.lax.dot_general(
        x, a_scratch[...], (((1,), (0,)), ((), ())),
        preferred_element_type=jnp.float32,
    )
    ids = ids_ref[...]
    lane = jax.lax.broadcasted_iota(jnp.int32, (BLOCK_N, E * R), 1)
    mask = (lane // R) == ids
    lr = jnp.where(mask, lr, 0.0)
    xl = jnp.concatenate([x, lr], axis=1)
    out = jax.lax.dot_general(
        xl, wb_scratch[...], (((1,), (0,)), ((), ())),
        preferred_element_type=jnp.float32,
    )
    out_ref[...] = out + bias_ref[...]


@jax.jit
def kernel(hidden_states, base_weight, base_bias, lora_a, lora_b, adapter_ids):
    a_flat = lora_a.reshape(E, D_IN * R)
    b_stacked = lora_b.reshape(E * R, D_OUT)
    ids2d = adapter_ids.astype(jnp.int32).reshape(N, 1)
    bias2d = base_bias.reshape(1, D_OUT)

    grid = (N // BLOCK_N,)
    out = pl.pallas_call(
        _fused_kernel,
        grid=grid,
        in_specs=[
            pl.BlockSpec((BLOCK_N, D_IN), lambda i: (i, 0)),
            pl.BlockSpec((D_IN, D_OUT), lambda i: (0, 0)),
            pl.BlockSpec((1, D_OUT), lambda i: (0, 0)),
            pl.BlockSpec((E, D_IN * R), lambda i: (0, 0)),
            pl.BlockSpec((E * R, D_OUT), lambda i: (0, 0)),
            pl.BlockSpec((BLOCK_N, 1), lambda i: (i, 0)),
        ],
        out_specs=pl.BlockSpec((BLOCK_N, D_OUT), lambda i: (i, 0)),
        out_shape=jax.ShapeDtypeStruct((N, D_OUT), jnp.float32),
        scratch_shapes=[
            pltpu.VMEM((D_IN + E * R, D_OUT), jnp.float32),
            pltpu.VMEM((D_IN, E * R), jnp.float32),
        ],
        compiler_params=pltpu.CompilerParams(
            dimension_semantics=("parallel",),
        ),
    )(hidden_states, base_weight, bias2d, a_flat, b_stacked, ids2d)
    return out


# speedup vs baseline: 1.1557x; 1.1557x over previous
import jax
import jax.numpy as jnp
from jax.experimental import pallas as pl
from jax.experimental.pallas import tpu as pltpu

N = 8192
D_IN = 768
D_OUT = 768
E = 16
R = 8
SCALING = 16.0 / 8.0

BLOCK_N = 1024


def _fused_kernel(x_ref, w_ref, bias_ref, a_ref, b_ref, ids_ref, out_ref,
                  wb_scratch):
    @pl.when(pl.program_id(0) == 0)
    def _build_wb():
        wb_scratch[:D_IN, :] = w_ref[...]
        wb_scratch[D_IN:, :] = b_ref[...] * SCALING

    x = x_ref[...]
    lr = jax.lax.dot_general(
        x, a_ref[...], (((1,), (0,)), ((), ())),
        preferred_element_type=jnp.float32,
    )
    ids = ids_ref[...]
    lane = jax.lax.broadcasted_iota(jnp.int32, (BLOCK_N, E * R), 1)
    mask = (lane // R) == ids
    lr = jnp.where(mask, lr, 0.0)
    xl = jnp.concatenate([x, lr], axis=1)
    out = jax.lax.dot_general(
        xl, wb_scratch[...], (((1,), (0,)), ((), ())),
        preferred_element_type=jnp.float32,
    )
    out_ref[...] = out + bias_ref[...]


@jax.jit
def kernel(hidden_states, base_weight, base_bias, lora_a, lora_b, adapter_ids):
    a_stacked = jnp.transpose(lora_a, (1, 0, 2)).reshape(D_IN, E * R)
    b_stacked = lora_b.reshape(E * R, D_OUT)
    ids2d = adapter_ids.astype(jnp.int32).reshape(N, 1)
    bias2d = base_bias.reshape(1, D_OUT)

    grid = (N // BLOCK_N,)
    out = pl.pallas_call(
        _fused_kernel,
        grid=grid,
        in_specs=[
            pl.BlockSpec((BLOCK_N, D_IN), lambda i: (i, 0)),
            pl.BlockSpec((D_IN, D_OUT), lambda i: (0, 0)),
            pl.BlockSpec((1, D_OUT), lambda i: (0, 0)),
            pl.BlockSpec((D_IN, E * R), lambda i: (0, 0)),
            pl.BlockSpec((E * R, D_OUT), lambda i: (0, 0)),
            pl.BlockSpec((BLOCK_N, 1), lambda i: (i, 0)),
        ],
        out_specs=pl.BlockSpec((BLOCK_N, D_OUT), lambda i: (i, 0)),
        out_shape=jax.ShapeDtypeStruct((N, D_OUT), jnp.float32),
        scratch_shapes=[pltpu.VMEM((D_IN + E * R, D_OUT), jnp.float32)],
        compiler_params=pltpu.CompilerParams(
            dimension_semantics=("parallel",),
        ),
    )(hidden_states, base_weight, bias2d, a_stacked, b_stacked, ids2d)
    return out


# speedup vs baseline: 1.1577x; 1.0017x over previous
import jax
import jax.numpy as jnp
from jax.experimental import pallas as pl
from jax.experimental.pallas import tpu as pltpu

N = 8192
D_IN = 768
D_OUT = 768
E = 16
R = 8
SCALING = 16.0 / 8.0

BLOCK_N = 1024


def _fused_kernel(x_ref, w_ref, bias_ref, a_ref, b_ref, ids_ref, out_ref,
                  wb_scratch):
    @pl.when(pl.program_id(0) == 0)
    def _build_wb():
        wb_scratch[:D_IN, :] = w_ref[...].astype(jnp.bfloat16)
        wb_scratch[D_IN:, :] = (b_ref[...] * SCALING).astype(jnp.bfloat16)

    x = x_ref[...].astype(jnp.bfloat16)
    lr = jax.lax.dot_general(
        x, a_ref[...], (((1,), (0,)), ((), ())),
        preferred_element_type=jnp.float32,
    )
    ids = ids_ref[...]
    lane = jax.lax.broadcasted_iota(jnp.int32, (BLOCK_N, E * R), 1)
    mask = (lane // R) == ids
    lr = jnp.where(mask, lr, 0.0).astype(jnp.bfloat16)
    xl = jnp.concatenate([x, lr], axis=1)
    out = jax.lax.dot_general(
        xl, wb_scratch[...], (((1,), (0,)), ((), ())),
        preferred_element_type=jnp.float32,
    )
    out_ref[...] = out + bias_ref[...]


@jax.jit
def kernel(hidden_states, base_weight, base_bias, lora_a, lora_b, adapter_ids):
    a_stacked = jnp.transpose(lora_a, (1, 0, 2)).reshape(D_IN, E * R).astype(jnp.bfloat16)
    b_stacked = lora_b.reshape(E * R, D_OUT)
    ids2d = adapter_ids.astype(jnp.int32).reshape(N, 1)
    bias2d = base_bias.reshape(1, D_OUT)

    grid = (N // BLOCK_N,)
    out = pl.pallas_call(
        _fused_kernel,
        grid=grid,
        in_specs=[
            pl.BlockSpec((BLOCK_N, D_IN), lambda i: (i, 0)),
            pl.BlockSpec((D_IN, D_OUT), lambda i: (0, 0)),
            pl.BlockSpec((1, D_OUT), lambda i: (0, 0)),
            pl.BlockSpec((D_IN, E * R), lambda i: (0, 0)),
            pl.BlockSpec((E * R, D_OUT), lambda i: (0, 0)),
            pl.BlockSpec((BLOCK_N, 1), lambda i: (i, 0)),
        ],
        out_specs=pl.BlockSpec((BLOCK_N, D_OUT), lambda i: (i, 0)),
        out_shape=jax.ShapeDtypeStruct((N, D_OUT), jnp.float32),
        scratch_shapes=[pltpu.VMEM((D_IN + E * R, D_OUT), jnp.bfloat16)],
        compiler_params=pltpu.CompilerParams(
            dimension_semantics=("parallel",),
        ),
    )(hidden_states, base_weight, bias2d, a_stacked, b_stacked, ids2d)
    return out
